# baseline (speedup 1.0000x reference)
import functools

import jax
import jax.numpy as jnp
from jax import lax
from jax.experimental import pallas as pl
from jax.experimental.pallas import tpu as pltpu
from jax.experimental.pallas import tpu_sc as plsc

_N = 10000
_E = 320000
_F_IN = 128
_HID = 16
_NC = 7

_NPAD = 10240
_NW = 32
_NS = 16
_RPS = _NPAD // _NS
_CW = 128
_NBUF = 4
_CHUNKS = 80
_GROUPS = _CHUNKS // _NBUF
_EPT = _CHUNKS * _CW
_EPAD = _EPT * _NW
_DEGQ = 8

_BLK = 256
_GRID = _NPAD // _BLK

_mesh = plsc.VectorSubcoreMesh(core_axis_name="c", subcore_axis_name="s")
_sc_params = pltpu.CompilerParams(use_tc_tiling_on_sc=False)


@functools.partial(
    pl.kernel,
    out_type=jax.ShapeDtypeStruct((2, _NPAD, _HID), jnp.float32),
    mesh=_mesh,
    scratch_types=[
        pltpu.VMEM((_CHUNKS, _CW), jnp.int32),
        pltpu.VMEM((_CW, _HID), jnp.float32),
        pltpu.VMEM_SHARED((_NPAD, _HID), jnp.float32),
    ] + [pltpu.SemaphoreType.DMA] * _DEGQ,
    compiler_params=_sc_params,
)
def _deg_kernel(dst_hbm, zeros_hbm, ones_hbm, out_hbm, idx_v, ones_v, acc, *sems):
    c = lax.axis_index("c")
    s = lax.axis_index("s")
    w = c * _NS + s
    pltpu.sync_copy(zeros_hbm, acc.at[pl.ds(s * _RPS, _RPS)])
    pltpu.sync_copy(ones_hbm, ones_v)
    pltpu.sync_copy(dst_hbm.at[w], idx_v)
    plsc.subcore_barrier()

    def scat(b, j):
        pltpu.async_copy(ones_v, acc.at[idx_v.at[j]], sems[b], add=True)

    def scat_wait(b, j):
        pltpu.make_async_copy(ones_v, acc.at[idx_v.at[j]], sems[b]).wait()

    for b in range(_DEGQ):
        scat(b, b)

    def body(g, carry):
        base = _DEGQ * g
        for b in range(_DEGQ):
            scat_wait(b, base + b)
            scat(b, base + _DEGQ + b)
        return carry

    ngrp = _CHUNKS // _DEGQ - 1
    lax.fori_loop(0, ngrp, body, 0)
    for b in range(_DEGQ):
        scat_wait(b, _CHUNKS - _DEGQ + b)
    plsc.subcore_barrier()
    pltpu.sync_copy(acc.at[pl.ds(s * _RPS, _RPS)],
                    out_hbm.at[c, pl.ds(s * _RPS, _RPS)])


@functools.partial(
    pl.kernel,
    out_type=jax.ShapeDtypeStruct((2, _NPAD, _HID), jnp.float32),
    mesh=_mesh,
    scratch_types=[
        pltpu.VMEM((_CHUNKS, _CW), jnp.int32),
        pltpu.VMEM((_CHUNKS, _CW), jnp.int32),
    ] + [pltpu.VMEM((_CW, _HID), jnp.float32)] * _NBUF
      + [pltpu.VMEM_SHARED((_NPAD, _HID), jnp.float32)]
      + [pltpu.SemaphoreType.DMA] * (2 * _NBUF),
    compiler_params=_sc_params,
)
def _agg_kernel(hs_hbm, src_hbm, dst_hbm, zeros_hbm, out_hbm,
                src_v, dst_v, *rest):
    rows = rest[:_NBUF]
    acc = rest[_NBUF]
    gsem = rest[_NBUF + 1:2 * _NBUF + 1]
    ssem = rest[2 * _NBUF + 1:]
    c = lax.axis_index("c")
    s = lax.axis_index("s")
    w = c * _NS + s
    pltpu.sync_copy(zeros_hbm, acc.at[pl.ds(s * _RPS, _RPS)])
    pltpu.sync_copy(src_hbm.at[w], src_v)
    pltpu.sync_copy(dst_hbm.at[w], dst_v)
    plsc.subcore_barrier()

    def gath(b, j):
        pltpu.async_copy(hs_hbm.at[src_v.at[j]], rows[b], gsem[b])

    def gath_wait(b, j):
        pltpu.make_async_copy(hs_hbm.at[src_v.at[j]], rows[b], gsem[b]).wait()

    def scat(b, j):
        pltpu.async_copy(rows[b], acc.at[dst_v.at[j]], ssem[b], add=True)

    def scat_wait(b, j):
        pltpu.make_async_copy(rows[b], acc.at[dst_v.at[j]], ssem[b]).wait()

    for b in range(_NBUF):
        gath(b, b)

    def body(g, carry):
        base = _NBUF * g
        for b in range(_NBUF):
            gath_wait(b, base + b)
            scat(b, base + b)

        @pl.when(g < _GROUPS - 1)
        def _():
            for b in range(_NBUF):
                scat_wait(b, base + b)
                gath(b, base + _NBUF + b)

        return carry

    lax.fori_loop(0, _GROUPS, body, 0)
    for b in range(_NBUF):
        scat_wait(b, _CHUNKS - _NBUF + b)
    plsc.subcore_barrier()
    pltpu.sync_copy(acc.at[pl.ds(s * _RPS, _RPS)],
                    out_hbm.at[c, pl.ds(s * _RPS, _RPS)])


def _dinv_block(dega):
    deg = dega[0, :, 0] + dega[1, :, 0] + 1.0
    return lax.rsqrt(deg)


def _row_scale(pid, dinv):
    rows = pid * _BLK + lax.broadcasted_iota(jnp.int32, (_BLK,), 0)
    return jnp.where(rows < _N, dinv, 0.0)


def _mm_body(x_ref, w1_ref, h1_ref):
    h1_ref[...] = jnp.dot(x_ref[...], w1_ref[...],
                          preferred_element_type=jnp.float32)


def _scale_body(h1_ref, dega_ref, hs1_ref):
    pid = pl.program_id(0)
    dinv = _dinv_block(dega_ref[...])
    hs1_ref[...] = h1_ref[...] * _row_scale(pid, dinv)[:, None]


def _phase_b_body(agg_ref, hs1_ref, dega_ref, w2_ref, b1_ref, hs2_ref):
    pid = pl.program_id(0)
    a = agg_ref[...]
    dinv = _dinv_block(dega_ref[...])
    out1 = jnp.maximum(dinv[:, None] * (a[0] + a[1] + hs1_ref[...]) + b1_ref[0], 0.0)
    h2 = jnp.dot(out1, w2_ref[...], preferred_element_type=jnp.float32)
    hs2_ref[...] = h2 * _row_scale(pid, dinv)[:, None]


def _phase_c_body(agg_ref, hs2_ref, dega_ref, b2_ref, out_ref):
    a = agg_ref[...]
    dinv = _dinv_block(dega_ref[...])
    logits = dinv[:, None] * (a[0] + a[1] + hs2_ref[...]) + b2_ref[0]
    col = lax.broadcasted_iota(jnp.int32, (_BLK, _HID), 1)
    valid = col < _NC
    ml = jnp.where(valid, logits, jnp.float32(-1e30))
    m = jnp.max(ml, axis=1, keepdims=True)
    e = jnp.where(valid, jnp.exp(ml - m), 0.0)
    lse = jnp.log(jnp.sum(e, axis=1, keepdims=True))
    out_ref[...] = logits - m - lse


_node_spec = pl.BlockSpec((_BLK, _HID), lambda i: (i, 0))
_pair_spec = pl.BlockSpec((2, _BLK, _HID), lambda i: (0, i, 0))
_f32 = jnp.float32


def _mm(x, w1):
    return pl.pallas_call(
        _mm_body,
        grid=(_GRID,),
        in_specs=[
            pl.BlockSpec((_BLK, _F_IN), lambda i: (i, 0)),
            pl.BlockSpec((_F_IN, _HID), lambda i: (0, 0)),
        ],
        out_specs=_node_spec,
        out_shape=jax.ShapeDtypeStruct((_NPAD, _HID), _f32),
    )(x, w1)


def _scale(h1, dega):
    return pl.pallas_call(
        _scale_body,
        grid=(_GRID,),
        in_specs=[_node_spec, _pair_spec],
        out_specs=_node_spec,
        out_shape=jax.ShapeDtypeStruct((_NPAD, _HID), _f32),
    )(h1, dega)


def _phase_b(agg1, hs1, dega, w2p, b1r):
    return pl.pallas_call(
        _phase_b_body,
        grid=(_GRID,),
        in_specs=[
            _pair_spec,
            _node_spec,
            _pair_spec,
            pl.BlockSpec((_HID, _HID), lambda i: (0, 0)),
            pl.BlockSpec((1, _HID), lambda i: (0, 0)),
        ],
        out_specs=_node_spec,
        out_shape=jax.ShapeDtypeStruct((_NPAD, _HID), _f32),
    )(agg1, hs1, dega, w2p, b1r)


def _phase_c(agg2, hs2, dega, b2r):
    return pl.pallas_call(
        _phase_c_body,
        grid=(_GRID,),
        in_specs=[
            _pair_spec,
            _node_spec,
            _pair_spec,
            pl.BlockSpec((1, _HID), lambda i: (0, 0)),
        ],
        out_specs=_node_spec,
        out_shape=jax.ShapeDtypeStruct((_NPAD, _HID), _f32),
    )(agg2, hs2, dega, b2r)


def kernel(x, edge_index, W1, b1, W2, b2):
    pad = _EPAD - _E
    pad_idx = _N + (jnp.arange(pad, dtype=jnp.int32) % (_NPAD - _N))
    src_p = jnp.concatenate([edge_index[0], pad_idx]).reshape(_NW, _CHUNKS, _CW)
    dst_p = jnp.concatenate([edge_index[1], pad_idx]).reshape(_NW, _CHUNKS, _CW)
    w2p = jnp.pad(W2, ((0, 0), (0, _HID - _NC)))
    b1r = b1.reshape(1, _HID)
    b2r = jnp.pad(b2, (0, _HID - _NC)).reshape(1, _HID)
    zeros_sub = jnp.zeros((_RPS, _HID), _f32)
    ones_cw = jnp.ones((_CW, _HID), _f32)

    dega = _deg_kernel(dst_p, zeros_sub, ones_cw)
    h1 = _mm(x, W1)
    hs1 = _scale(h1, dega)
    agg1 = _agg_kernel(hs1, src_p, dst_p, zeros_sub)
    hs2 = _phase_b(agg1, hs1, dega, w2p, b1r)
    agg2 = _agg_kernel(hs2, src_p, dst_p, zeros_sub)
    out = _phase_c(agg2, hs2, dega, b2r)
    return out[:_N, :_NC]


# speedup vs baseline: 1.1328x; 1.1328x over previous
import functools

import jax
import jax.numpy as jnp
from jax import lax
from jax.experimental import pallas as pl
from jax.experimental.pallas import tpu as pltpu
from jax.experimental.pallas import tpu_sc as plsc

_N = 10000
_E = 320000
_F_IN = 128
_HID = 16
_NC = 7

_NPAD = 10240
_NW = 32
_NS = 16
_RPS = _NPAD // _NS
_CW = 128
_NBUF = 8
_CHUNKS = 80
_GROUPS = _CHUNKS // _NBUF
_EPT = _CHUNKS * _CW
_EPAD = _EPT * _NW
_DEGQ = 8

_BLK = 256
_GRID = _NPAD // _BLK

_mesh = plsc.VectorSubcoreMesh(core_axis_name="c", subcore_axis_name="s")
_sc_params = pltpu.CompilerParams(use_tc_tiling_on_sc=False)


@functools.partial(
    pl.kernel,
    out_type=jax.ShapeDtypeStruct((2, _NPAD, _HID), jnp.float32),
    mesh=_mesh,
    scratch_types=[
        pltpu.VMEM((_CHUNKS, _CW), jnp.int32),
        pltpu.VMEM((_CW, _HID), jnp.float32),
        pltpu.VMEM_SHARED((_NPAD, _HID), jnp.float32),
    ] + [pltpu.SemaphoreType.DMA] * _DEGQ,
    compiler_params=_sc_params,
)
def _deg_kernel(dst_hbm, zeros_hbm, ones_hbm, out_hbm, idx_v, ones_v, acc, *sems):
    c = lax.axis_index("c")
    s = lax.axis_index("s")
    w = c * _NS + s
    d0 = pltpu.async_copy(zeros_hbm, acc.at[pl.ds(s * _RPS, _RPS)], sems[0])
    d1 = pltpu.async_copy(ones_hbm, ones_v, sems[1])
    d2 = pltpu.async_copy(dst_hbm.at[w], idx_v, sems[2])
    d0.wait()
    d1.wait()
    d2.wait()
    plsc.subcore_barrier()

    def scat(b, j):
        pltpu.async_copy(ones_v, acc.at[idx_v.at[j]], sems[b], add=True)

    def scat_wait(b, j):
        pltpu.make_async_copy(ones_v, acc.at[idx_v.at[j]], sems[b]).wait()

    for b in range(_DEGQ):
        scat(b, b)

    def body(g, carry):
        base = _DEGQ * g
        for b in range(_DEGQ):
            scat_wait(b, base + b)
            scat(b, base + _DEGQ + b)
        return carry

    ngrp = _CHUNKS // _DEGQ - 1
    lax.fori_loop(0, ngrp, body, 0)
    for b in range(_DEGQ):
        scat_wait(b, _CHUNKS - _DEGQ + b)
    plsc.subcore_barrier()
    pltpu.sync_copy(acc.at[pl.ds(s * _RPS, _RPS)],
                    out_hbm.at[c, pl.ds(s * _RPS, _RPS)])


@functools.partial(
    pl.kernel,
    out_type=jax.ShapeDtypeStruct((2, _NPAD, _HID), jnp.float32),
    mesh=_mesh,
    scratch_types=[
        pltpu.VMEM((_CHUNKS, _CW), jnp.int32),
        pltpu.VMEM((_CHUNKS, _CW), jnp.int32),
    ] + [pltpu.VMEM((_CW, _HID), jnp.float32)] * _NBUF
      + [pltpu.VMEM_SHARED((_NPAD, _HID), jnp.float32)]
      + [pltpu.SemaphoreType.DMA] * (2 * _NBUF),
    compiler_params=_sc_params,
)
def _agg_kernel(hs_hbm, src_hbm, dst_hbm, zeros_hbm, out_hbm,
                src_v, dst_v, *rest):
    rows = rest[:_NBUF]
    acc = rest[_NBUF]
    gsem = rest[_NBUF + 1:2 * _NBUF + 1]
    ssem = rest[2 * _NBUF + 1:]
    c = lax.axis_index("c")
    s = lax.axis_index("s")
    w = c * _NS + s
    d0 = pltpu.async_copy(zeros_hbm, acc.at[pl.ds(s * _RPS, _RPS)], gsem[0])
    d1 = pltpu.async_copy(src_hbm.at[w], src_v, gsem[1])
    d2 = pltpu.async_copy(dst_hbm.at[w], dst_v, gsem[2])
    d0.wait()
    d1.wait()
    d2.wait()
    plsc.subcore_barrier()

    def gath(b, j):
        pltpu.async_copy(hs_hbm.at[src_v.at[j]], rows[b], gsem[b])

    def gath_wait(b, j):
        pltpu.make_async_copy(hs_hbm.at[src_v.at[j]], rows[b], gsem[b]).wait()

    def scat(b, j):
        pltpu.async_copy(rows[b], acc.at[dst_v.at[j]], ssem[b], add=True)

    def scat_wait(b, j):
        pltpu.make_async_copy(rows[b], acc.at[dst_v.at[j]], ssem[b]).wait()

    for b in range(_NBUF):
        gath(b, b)

    def body(g, carry):
        base = _NBUF * g
        for b in range(_NBUF):
            gath_wait(b, base + b)
            scat(b, base + b)

        @pl.when(g < _GROUPS - 1)
        def _():
            for b in range(_NBUF):
                scat_wait(b, base + b)
                gath(b, base + _NBUF + b)

        return carry

    lax.fori_loop(0, _GROUPS, body, 0)
    for b in range(_NBUF):
        scat_wait(b, _CHUNKS - _NBUF + b)
    plsc.subcore_barrier()
    pltpu.sync_copy(acc.at[pl.ds(s * _RPS, _RPS)],
                    out_hbm.at[c, pl.ds(s * _RPS, _RPS)])


def _dinv_block(dega):
    deg = dega[0, :, 0] + dega[1, :, 0] + 1.0
    return lax.rsqrt(deg)


def _row_scale(pid, dinv):
    rows = pid * _BLK + lax.broadcasted_iota(jnp.int32, (_BLK,), 0)
    return jnp.where(rows < _N, dinv, 0.0)


def _phase_a_body(x_ref, w1_ref, dega_ref, hs1_ref):
    pid = pl.program_id(0)
    h = jnp.dot(x_ref[...], w1_ref[...], preferred_element_type=jnp.float32)
    dinv = _dinv_block(dega_ref[...])
    hs1_ref[...] = h * _row_scale(pid, dinv)[:, None]


def _phase_b_body(agg_ref, hs1_ref, dega_ref, w2_ref, b1_ref, hs2_ref):
    pid = pl.program_id(0)
    a = agg_ref[...]
    dinv = _dinv_block(dega_ref[...])
    out1 = jnp.maximum(dinv[:, None] * (a[0] + a[1] + hs1_ref[...]) + b1_ref[0], 0.0)
    h2 = jnp.dot(out1, w2_ref[...], preferred_element_type=jnp.float32)
    hs2_ref[...] = h2 * _row_scale(pid, dinv)[:, None]


def _phase_c_body(agg_ref, hs2_ref, dega_ref, b2_ref, out_ref):
    a = agg_ref[...]
    dinv = _dinv_block(dega_ref[...])
    logits = dinv[:, None] * (a[0] + a[1] + hs2_ref[...]) + b2_ref[0]
    col = lax.broadcasted_iota(jnp.int32, (_BLK, _HID), 1)
    valid = col < _NC
    ml = jnp.where(valid, logits, jnp.float32(-1e30))
    m = jnp.max(ml, axis=1, keepdims=True)
    e = jnp.where(valid, jnp.exp(ml - m), 0.0)
    lse = jnp.log(jnp.sum(e, axis=1, keepdims=True))
    out_ref[...] = logits - m - lse


_node_spec = pl.BlockSpec((_BLK, _HID), lambda i: (i, 0))
_pair_spec = pl.BlockSpec((2, _BLK, _HID), lambda i: (0, i, 0))
_f32 = jnp.float32


def _phase_a(x, w1, dega):
    return pl.pallas_call(
        _phase_a_body,
        grid=(_GRID,),
        in_specs=[
            pl.BlockSpec((_BLK, _F_IN), lambda i: (i, 0)),
            pl.BlockSpec((_F_IN, _HID), lambda i: (0, 0)),
            _pair_spec,
        ],
        out_specs=_node_spec,
        out_shape=jax.ShapeDtypeStruct((_NPAD, _HID), _f32),
    )(x, w1, dega)


def _phase_b(agg1, hs1, dega, w2p, b1r):
    return pl.pallas_call(
        _phase_b_body,
        grid=(_GRID,),
        in_specs=[
            _pair_spec,
            _node_spec,
            _pair_spec,
            pl.BlockSpec((_HID, _HID), lambda i: (0, 0)),
            pl.BlockSpec((1, _HID), lambda i: (0, 0)),
        ],
        out_specs=_node_spec,
        out_shape=jax.ShapeDtypeStruct((_NPAD, _HID), _f32),
    )(agg1, hs1, dega, w2p, b1r)


def _phase_c(agg2, hs2, dega, b2r):
    return pl.pallas_call(
        _phase_c_body,
        grid=(_GRID,),
        in_specs=[
            _pair_spec,
            _node_spec,
            _pair_spec,
            pl.BlockSpec((1, _HID), lambda i: (0, 0)),
        ],
        out_specs=_node_spec,
        out_shape=jax.ShapeDtypeStruct((_NPAD, _HID), _f32),
    )(agg2, hs2, dega, b2r)


def kernel(x, edge_index, W1, b1, W2, b2):
    pad = _EPAD - _E
    pad_idx = _N + (jnp.arange(pad, dtype=jnp.int32) % (_NPAD - _N))
    src_p = jnp.concatenate([edge_index[0], pad_idx]).reshape(_NW, _CHUNKS, _CW)
    dst_p = jnp.concatenate([edge_index[1], pad_idx]).reshape(_NW, _CHUNKS, _CW)
    w2p = jnp.pad(W2, ((0, 0), (0, _HID - _NC)))
    b1r = b1.reshape(1, _HID)
    b2r = jnp.pad(b2, (0, _HID - _NC)).reshape(1, _HID)
    zeros_sub = jnp.zeros((_RPS, _HID), _f32)
    ones_cw = jnp.ones((_CW, _HID), _f32)

    dega = _deg_kernel(dst_p, zeros_sub, ones_cw)
    hs1 = _phase_a(x, W1, dega)
    agg1 = _agg_kernel(hs1, src_p, dst_p, zeros_sub)
    hs2 = _phase_b(agg1, hs1, dega, w2p, b1r)
    agg2 = _agg_kernel(hs2, src_p, dst_p, zeros_sub)
    out = _phase_c(agg2, hs2, dega, b2r)
    return out[:_N, :_NC]


# speedup vs baseline: 1.5179x; 1.3400x over previous
import functools

import jax
import jax.numpy as jnp
from jax import lax
from jax.experimental import pallas as pl
from jax.experimental.pallas import tpu as pltpu
from jax.experimental.pallas import tpu_sc as plsc

_N = 10000
_E = 320000
_F_IN = 128
_HID = 16
_NC = 7

_NPAD = 10240
_NW = 32
_NS = 16
_RPS = _NPAD // _NS
_CW = 128
_NBUF = 8
_CHUNKS = 80
_GROUPS = _CHUNKS // _NBUF
_EPT = _CHUNKS * _CW
_EPAD = _EPT * _NW
_DEGQ = 8

_BLK = 2048
_GRID = _NPAD // _BLK

_mesh = plsc.VectorSubcoreMesh(core_axis_name="c", subcore_axis_name="s")
_sc_params = pltpu.CompilerParams(use_tc_tiling_on_sc=False)


@functools.partial(
    pl.kernel,
    out_type=jax.ShapeDtypeStruct((2, _NPAD, _HID), jnp.float32),
    mesh=_mesh,
    scratch_types=[
        pltpu.VMEM((_CHUNKS, _CW), jnp.int32),
        pltpu.VMEM((_CW, _HID), jnp.float32),
        pltpu.VMEM_SHARED((_NPAD, _HID), jnp.float32),
    ] + [pltpu.SemaphoreType.DMA] * _DEGQ,
    compiler_params=_sc_params,
)
def _deg_kernel(dst_hbm, zeros_hbm, ones_hbm, out_hbm, idx_v, ones_v, acc, *sems):
    c = lax.axis_index("c")
    s = lax.axis_index("s")
    w = c * _NS + s
    d0 = pltpu.async_copy(zeros_hbm, acc.at[pl.ds(s * _RPS, _RPS)], sems[0])
    d1 = pltpu.async_copy(ones_hbm, ones_v, sems[1])
    d2 = pltpu.async_copy(dst_hbm.at[w], idx_v, sems[2])
    d0.wait()
    d1.wait()
    d2.wait()
    plsc.subcore_barrier()

    def scat(b, j):
        pltpu.async_copy(ones_v, acc.at[idx_v.at[j]], sems[b], add=True)

    def scat_wait(b, j):
        pltpu.make_async_copy(ones_v, acc.at[idx_v.at[j]], sems[b]).wait()

    for b in range(_DEGQ):
        scat(b, b)

    def body(g, carry):
        base = _DEGQ * g
        for b in range(_DEGQ):
            scat_wait(b, base + b)
            scat(b, base + _DEGQ + b)
        return carry

    ngrp = _CHUNKS // _DEGQ - 1
    lax.fori_loop(0, ngrp, body, 0)
    for b in range(_DEGQ):
        scat_wait(b, _CHUNKS - _DEGQ + b)
    plsc.subcore_barrier()
    pltpu.sync_copy(acc.at[pl.ds(s * _RPS, _RPS)],
                    out_hbm.at[c, pl.ds(s * _RPS, _RPS)])


@functools.partial(
    pl.kernel,
    out_type=jax.ShapeDtypeStruct((2, _NPAD, _HID), jnp.float32),
    mesh=_mesh,
    scratch_types=[
        pltpu.VMEM((_CHUNKS, _CW), jnp.int32),
        pltpu.VMEM((_CHUNKS, _CW), jnp.int32),
    ] + [pltpu.VMEM((_CW, _HID), jnp.float32)] * _NBUF
      + [pltpu.VMEM_SHARED((_NPAD, _HID), jnp.float32)]
      + [pltpu.SemaphoreType.DMA] * (2 * _NBUF),
    compiler_params=_sc_params,
)
def _agg_kernel(hs_hbm, src_hbm, dst_hbm, zeros_hbm, out_hbm,
                src_v, dst_v, *rest):
    rows = rest[:_NBUF]
    acc = rest[_NBUF]
    gsem = rest[_NBUF + 1:2 * _NBUF + 1]
    ssem = rest[2 * _NBUF + 1:]
    c = lax.axis_index("c")
    s = lax.axis_index("s")
    w = c * _NS + s
    d0 = pltpu.async_copy(zeros_hbm, acc.at[pl.ds(s * _RPS, _RPS)], gsem[0])
    d1 = pltpu.async_copy(src_hbm.at[w], src_v, gsem[1])
    d2 = pltpu.async_copy(dst_hbm.at[w], dst_v, gsem[2])
    d0.wait()
    d1.wait()
    d2.wait()
    plsc.subcore_barrier()

    def gath(b, j):
        pltpu.async_copy(hs_hbm.at[src_v.at[j]], rows[b], gsem[b])

    def gath_wait(b, j):
        pltpu.make_async_copy(hs_hbm.at[src_v.at[j]], rows[b], gsem[b]).wait()

    def scat(b, j):
        pltpu.async_copy(rows[b], acc.at[dst_v.at[j]], ssem[b], add=True)

    def scat_wait(b, j):
        pltpu.make_async_copy(rows[b], acc.at[dst_v.at[j]], ssem[b]).wait()

    for b in range(_NBUF):
        gath(b, b)

    def body(g, carry):
        base = _NBUF * g
        for b in range(_NBUF):
            gath_wait(b, base + b)
            scat(b, base + b)

        @pl.when(g < _GROUPS - 1)
        def _():
            for b in range(_NBUF):
                scat_wait(b, base + b)
                gath(b, base + _NBUF + b)

        return carry

    lax.fori_loop(0, _GROUPS, body, 0)
    for b in range(_NBUF):
        scat_wait(b, _CHUNKS - _NBUF + b)
    plsc.subcore_barrier()
    pltpu.sync_copy(acc.at[pl.ds(s * _RPS, _RPS)],
                    out_hbm.at[c, pl.ds(s * _RPS, _RPS)])


def _dinv_block(dega):
    deg = dega[0, :, 0] + dega[1, :, 0] + 1.0
    return lax.rsqrt(deg)


def _row_scale(pid, dinv):
    rows = pid * _BLK + lax.broadcasted_iota(jnp.int32, (_BLK,), 0)
    return jnp.where(rows < _N, dinv, 0.0)


def _phase_a_body(x_ref, w1_ref, dega_ref, hs1_ref):
    pid = pl.program_id(0)
    h = jnp.dot(x_ref[...], w1_ref[...], preferred_element_type=jnp.float32)
    dinv = _dinv_block(dega_ref[...])
    hs1_ref[...] = h * _row_scale(pid, dinv)[:, None]


def _phase_b_body(agg_ref, hs1_ref, dega_ref, w2_ref, b1_ref, hs2_ref):
    pid = pl.program_id(0)
    a = agg_ref[...]
    dinv = _dinv_block(dega_ref[...])
    out1 = jnp.maximum(dinv[:, None] * (a[0] + a[1] + hs1_ref[...]) + b1_ref[0], 0.0)
    h2 = jnp.dot(out1, w2_ref[...], preferred_element_type=jnp.float32)
    hs2_ref[...] = h2 * _row_scale(pid, dinv)[:, None]


def _phase_c_body(agg_ref, hs2_ref, dega_ref, b2_ref, out_ref):
    a = agg_ref[...]
    dinv = _dinv_block(dega_ref[...])
    logits = dinv[:, None] * (a[0] + a[1] + hs2_ref[...]) + b2_ref[0]
    col = lax.broadcasted_iota(jnp.int32, (_BLK, _HID), 1)
    valid = col < _NC
    ml = jnp.where(valid, logits, jnp.float32(-1e30))
    m = jnp.max(ml, axis=1, keepdims=True)
    e = jnp.where(valid, jnp.exp(ml - m), 0.0)
    lse = jnp.log(jnp.sum(e, axis=1, keepdims=True))
    out_ref[...] = logits - m - lse


_node_spec = pl.BlockSpec((_BLK, _HID), lambda i: (i, 0))
_pair_spec = pl.BlockSpec((2, _BLK, _HID), lambda i: (0, i, 0))
_f32 = jnp.float32


def _phase_a(x, w1, dega):
    return pl.pallas_call(
        _phase_a_body,
        grid=(_GRID,),
        in_specs=[
            pl.BlockSpec((_BLK, _F_IN), lambda i: (i, 0)),
            pl.BlockSpec((_F_IN, _HID), lambda i: (0, 0)),
            _pair_spec,
        ],
        out_specs=_node_spec,
        out_shape=jax.ShapeDtypeStruct((_NPAD, _HID), _f32),
    )(x, w1, dega)


def _phase_b(agg1, hs1, dega, w2p, b1r):
    return pl.pallas_call(
        _phase_b_body,
        grid=(_GRID,),
        in_specs=[
            _pair_spec,
            _node_spec,
            _pair_spec,
            pl.BlockSpec((_HID, _HID), lambda i: (0, 0)),
            pl.BlockSpec((1, _HID), lambda i: (0, 0)),
        ],
        out_specs=_node_spec,
        out_shape=jax.ShapeDtypeStruct((_NPAD, _HID), _f32),
    )(agg1, hs1, dega, w2p, b1r)


def _phase_c(agg2, hs2, dega, b2r):
    return pl.pallas_call(
        _phase_c_body,
        grid=(_GRID,),
        in_specs=[
            _pair_spec,
            _node_spec,
            _pair_spec,
            pl.BlockSpec((1, _HID), lambda i: (0, 0)),
        ],
        out_specs=_node_spec,
        out_shape=jax.ShapeDtypeStruct((_NPAD, _HID), _f32),
    )(agg2, hs2, dega, b2r)


def kernel(x, edge_index, W1, b1, W2, b2):
    pad = _EPAD - _E
    pad_idx = _N + (jnp.arange(pad, dtype=jnp.int32) % (_NPAD - _N))
    src_p = jnp.concatenate([edge_index[0], pad_idx]).reshape(_NW, _CHUNKS, _CW)
    dst_p = jnp.concatenate([edge_index[1], pad_idx]).reshape(_NW, _CHUNKS, _CW)
    w2p = jnp.pad(W2, ((0, 0), (0, _HID - _NC)))
    b1r = b1.reshape(1, _HID)
    b2r = jnp.pad(b2, (0, _HID - _NC)).reshape(1, _HID)
    zeros_sub = jnp.zeros((_RPS, _HID), _f32)
    ones_cw = jnp.ones((_CW, _HID), _f32)

    dega = _deg_kernel(dst_p, zeros_sub, ones_cw)
    hs1 = _phase_a(x, W1, dega)
    agg1 = _agg_kernel(hs1, src_p, dst_p, zeros_sub)
    hs2 = _phase_b(agg1, hs1, dega, w2p, b1r)
    agg2 = _agg_kernel(hs2, src_p, dst_p, zeros_sub)
    out = _phase_c(agg2, hs2, dega, b2r)
    return out[:_N, :_NC]
